# Initial kernel scaffold; baseline (speedup 1.0000x reference)
import jax
import jax.numpy as jnp
from jax.experimental import pallas as pl


def kernel(x, edge_index, edge_attr, batch, batch_size, params):
    raise NotImplementedError("write your pallas kernel here")


# speedup vs baseline: 1.3668x; 1.3668x over previous
import functools

import jax
import jax.numpy as jnp
from jax import lax
from jax.experimental import pallas as pl
from jax.experimental.pallas import tpu as pltpu
from jax.experimental.pallas import tpu_sc as plsc

NC = 2
NS = 16
NW = NC * NS
CHUNK = 128

_PREC = lax.Precision.HIGHEST
_MLP_PREC = lax.Precision.DEFAULT


def _gather_rows(h, src_p, per_w):
    e_pad = src_p.shape[0]
    n, c = h.shape
    mesh = plsc.VectorSubcoreMesh(core_axis_name="c", subcore_axis_name="s")

    @functools.partial(
        pl.kernel,
        out_type=jax.ShapeDtypeStruct((e_pad, c), jnp.float32),
        mesh=mesh,
        scratch_types=[
            pltpu.VMEM((CHUNK, c), jnp.float32),
            pltpu.VMEM((CHUNK, c), jnp.float32),
            pltpu.VMEM((CHUNK,), jnp.int32),
            pltpu.VMEM((CHUNK,), jnp.int32),
            pltpu.SemaphoreType.DMA,
            pltpu.SemaphoreType.DMA,
            pltpu.SemaphoreType.DMA,
            pltpu.SemaphoreType.DMA,
            pltpu.SemaphoreType.DMA,
        ],
    )
    def k(h_hbm, s_hbm, g_hbm, rb0, rb1, ib0, ib1, si0, si1, sg, so0, so1):
        w = lax.axis_index("s") * NC + lax.axis_index("c")
        base = w * per_w * CHUNK
        rb = (rb0, rb1)
        ib = (ib0, ib1)
        si = (si0, si1)
        so = (so0, so1)

        def idx_copy(b, kk):
            return pltpu.make_async_copy(
                s_hbm.at[pl.ds(base + kk * CHUNK, CHUNK)], ib[b], si[b])

        def out_copy(b, kk):
            return pltpu.make_async_copy(
                rb[b], g_hbm.at[pl.ds(base + kk * CHUNK, CHUNK)], so[b])

        idx_copy(0, 0).start()
        idx_copy(1, 1).start()

        @pl.loop(0, per_w, step=2)
        def _(k0):
            for b in range(2):
                kk = k0 + b
                idx_copy(b, kk).wait()

                @pl.when(kk >= 2)
                def _():
                    out_copy(b, kk).wait()

                pltpu.async_copy(h_hbm.at[ib[b]], rb[b], sg).wait()
                out_copy(b, kk).start()

                @pl.when(kk + 2 < per_w)
                def _():
                    idx_copy(b, kk + 2).start()

        out_copy(0, 0).wait()
        out_copy(1, 1).wait()

    return k(h, src_p)


def _segment_add(msg, dst_p, zeros_blk, aggr_rows, per_w):
    e_pad = dst_p.shape[0]
    c_dim = msg.shape[1]
    rpt = aggr_rows // NS
    mesh = plsc.VectorSubcoreMesh(core_axis_name="c", subcore_axis_name="s")

    @functools.partial(
        pl.kernel,
        out_type=jax.ShapeDtypeStruct((2 * aggr_rows, c_dim), jnp.float32),
        mesh=mesh,
        scratch_types=[
            pltpu.VMEM((CHUNK, c_dim), jnp.float32),
            pltpu.VMEM((CHUNK, c_dim), jnp.float32),
            pltpu.VMEM((CHUNK,), jnp.int32),
            pltpu.VMEM((CHUNK,), jnp.int32),
            pltpu.VMEM_SHARED((aggr_rows, c_dim), jnp.float32),
            pltpu.SemaphoreType.DMA,
            pltpu.SemaphoreType.DMA,
            pltpu.SemaphoreType.DMA,
            pltpu.SemaphoreType.DMA,
            pltpu.SemaphoreType.DMA,
        ],
    )
    def k(m_hbm, d_hbm, z_hbm, o_hbm, mb0, mb1, ib0, ib1, acc,
          sm0, sm1, si0, si1, sz):
        cc = lax.axis_index("c")
        ss = lax.axis_index("s")
        w = ss * NC + cc
        base = w * per_w * CHUNK
        r0 = ss * rpt
        pltpu.async_copy(z_hbm, acc.at[pl.ds(r0, rpt)], sz).wait()
        plsc.subcore_barrier()
        mb = (mb0, mb1)
        ib = (ib0, ib1)
        sm = (sm0, sm1)
        si = (si0, si1)

        def msg_copy(b, kk):
            return pltpu.make_async_copy(
                m_hbm.at[pl.ds(base + kk * CHUNK, CHUNK)], mb[b], sm[b])

        def idx_copy(b, kk):
            return pltpu.make_async_copy(
                d_hbm.at[pl.ds(base + kk * CHUNK, CHUNK)], ib[b], si[b])

        msg_copy(0, 0).start()
        idx_copy(0, 0).start()
        msg_copy(1, 1).start()
        idx_copy(1, 1).start()

        @pl.loop(0, per_w, step=2)
        def _(k0):
            for b in range(2):
                kk = k0 + b
                msg_copy(b, kk).wait()
                idx_copy(b, kk).wait()
                pltpu.sync_copy(mb[b], acc.at[ib[b]], add=True)

                @pl.when(kk + 2 < per_w)
                def _():
                    msg_copy(b, kk + 2).start()
                    idx_copy(b, kk + 2).start()

        plsc.subcore_barrier()
        pltpu.sync_copy(acc.at[pl.ds(r0, rpt)],
                        o_hbm.at[pl.ds(cc * aggr_rows + r0, rpt)])

    return k(msg, dst_p, zeros_blk)


def _edge_msg(g, ea0, ea1, wcat):
    e_pad, c = g.shape
    blk = 1024
    grid = e_pad // blk

    def body(g_ref, a0_ref, a1_ref, w_ref, o_ref):
        w = w_ref[...]
        t = (a0_ref[...] * w[0:1, :] + a1_ref[...] * w[1:2, :]) + g_ref[...]
        o_ref[...] = jnp.maximum(t + w[2:3, :], 0.0)

    return pl.pallas_call(
        body,
        grid=(grid,),
        in_specs=[
            pl.BlockSpec((blk, c), lambda i: (i, 0)),
            pl.BlockSpec((blk, 1), lambda i: (i, 0)),
            pl.BlockSpec((blk, 1), lambda i: (i, 0)),
            pl.BlockSpec((8, c), lambda i: (0, 0)),
        ],
        out_specs=pl.BlockSpec((blk, c), lambda i: (i, 0)),
        out_shape=jax.ShapeDtypeStruct((e_pad, c), jnp.float32),
    )(g, ea0, ea1, wcat)


def _stats(t):
    return (jnp.mean(t, axis=0, keepdims=True),
            jnp.var(t, axis=0, keepdims=True))


def _mm_first(h, aggr, aggr_rows, onep, w, b):
    n, c = h.shape

    def body(h_ref, ag_ref, onep_ref, w_ref, b_ref, o_ref):
        a = ag_ref[0:n, :] + ag_ref[aggr_rows:aggr_rows + n, :]
        u = onep_ref[...] * h_ref[...] + a
        o_ref[...] = jnp.dot(u, w_ref[...], precision=_MLP_PREC,
                             preferred_element_type=jnp.float32) + b_ref[...]

    return pl.pallas_call(
        body, out_shape=jax.ShapeDtypeStruct((n, c), jnp.float32),
    )(h, aggr, onep, w, b.reshape(1, c))


def _bn_mm(t, mu, var, g, be, w, b):
    n, c = t.shape

    def body(t_ref, mu_ref, var_ref, g_ref, be_ref, w_ref, b_ref, o_ref):
        tt = t_ref[...]
        z = jnp.maximum((tt - mu_ref[...]) / jnp.sqrt(var_ref[...] + 1e-5)
                        * g_ref[...] + be_ref[...], 0.0)
        o_ref[...] = jnp.dot(z, w_ref[...], precision=_MLP_PREC,
                             preferred_element_type=jnp.float32) + b_ref[...]

    return pl.pallas_call(
        body, out_shape=jax.ShapeDtypeStruct((n, c), jnp.float32),
    )(t, mu, var, g.reshape(1, c), be.reshape(1, c), w, b.reshape(1, c))


def _bn_mm_relu(t, mu, var, g, be, w, b):
    n, c = t.shape

    def body(t_ref, mu_ref, var_ref, g_ref, be_ref, w_ref, b_ref, o_ref):
        tt = t_ref[...]
        z = jnp.maximum((tt - mu_ref[...]) / jnp.sqrt(var_ref[...] + 1e-5)
                        * g_ref[...] + be_ref[...], 0.0)
        t3 = jnp.dot(z, w_ref[...], precision=_MLP_PREC,
                     preferred_element_type=jnp.float32) + b_ref[...]
        o_ref[...] = jnp.maximum(t3, 0.0)

    return pl.pallas_call(
        body, out_shape=jax.ShapeDtypeStruct((n, c), jnp.float32),
    )(t, mu, var, g.reshape(1, c), be.reshape(1, c), w, b.reshape(1, c))


def _node_mlp(h, aggr, aggr_rows, cp):
    n, c = h.shape
    onep = (1.0 + cp["eps"]).reshape(1, 1).astype(jnp.float32)
    (w1, b1), (w2, b2), (w3, b3) = cp["mlp"]["lins"]
    (g1, be1), (g2, be2) = cp["mlp"]["norms"]
    t1 = _mm_first(h, aggr, aggr_rows, onep, w1, b1)
    mu1, var1 = _stats(t1)
    t2 = _bn_mm(t1, mu1, var1, g1, be1, w2, b2)
    mu2, var2 = _stats(t2)
    return _bn_mm_relu(t2, mu2, var2, g2, be2, w3, b3)


def _pool_mm(reps, batch2, bs, ws, bss):
    n, c = reps[0].shape

    def body(*refs):
        r_refs = refs[:4]
        b_ref = refs[4]
        w_refs = refs[5:9]
        bb_refs = refs[9:13]
        o_refs = refs[13:]
        ids = b_ref[...]
        oht = (lax.broadcasted_iota(jnp.int32, (bs, n), 0) == ids
               ).astype(jnp.float32)
        for l in range(4):
            pooled = jnp.dot(oht, r_refs[l][...], precision=_PREC,
                             preferred_element_type=jnp.float32)
            o_refs[l][...] = jnp.dot(pooled, w_refs[l][...], precision=_MLP_PREC,
                                     preferred_element_type=jnp.float32) + bb_refs[l][...]

    return pl.pallas_call(
        body,
        out_shape=[jax.ShapeDtypeStruct((bs, c), jnp.float32)] * 4,
    )(*reps, batch2, *ws, *[b.reshape(1, c) for b in bss])


def _bn_mm4(ts, stats, gs, bes, ws, bs_):
    n, c = ts[0].shape

    def body(*refs):
        t_refs = refs[0:4]
        mu_refs = refs[4:8]
        var_refs = refs[8:12]
        g_refs = refs[12:16]
        be_refs = refs[16:20]
        w_refs = refs[20:24]
        b_refs = refs[24:28]
        o_refs = refs[28:]
        for l in range(4):
            tt = t_refs[l][...]
            z = jnp.maximum((tt - mu_refs[l][...]) / jnp.sqrt(var_refs[l][...] + 1e-5)
                            * g_refs[l][...] + be_refs[l][...], 0.0)
            o_refs[l][...] = jnp.dot(z, w_refs[l][...], precision=_MLP_PREC,
                                     preferred_element_type=jnp.float32) + b_refs[l][...]

    mus = [s[0] for s in stats]
    vars_ = [s[1] for s in stats]
    return pl.pallas_call(
        body,
        out_shape=[jax.ShapeDtypeStruct((n, c), jnp.float32)] * 4,
    )(*ts, *mus, *vars_, *[g.reshape(1, c) for g in gs],
      *[b.reshape(1, c) for b in bes], *ws, *[b.reshape(1, c) for b in bs_])


def _bn_sum4(ts, stats, gs, bes, ws, bs_):
    n, c = ts[0].shape

    def body(*refs):
        t_refs = refs[0:4]
        mu_refs = refs[4:8]
        var_refs = refs[8:12]
        g_refs = refs[12:16]
        be_refs = refs[16:20]
        w_refs = refs[20:24]
        b_refs = refs[24:28]
        o_ref = refs[-1]
        acc = None
        for l in range(4):
            tt = t_refs[l][...]
            z = jnp.maximum((tt - mu_refs[l][...]) / jnp.sqrt(var_refs[l][...] + 1e-5)
                            * g_refs[l][...] + be_refs[l][...], 0.0)
            zz = jnp.dot(z, w_refs[l][...], precision=_MLP_PREC,
                         preferred_element_type=jnp.float32) + b_refs[l][...]
            acc = zz if acc is None else acc + zz
        o_ref[...] = acc

    mus = [s[0] for s in stats]
    vars_ = [s[1] for s in stats]
    return pl.pallas_call(
        body,
        out_shape=jax.ShapeDtypeStruct((n, c), jnp.float32),
    )(*ts, *mus, *vars_, *[g.reshape(1, c) for g in gs],
      *[b.reshape(1, c) for b in bes], *ws, *[b.reshape(1, c) for b in bs_])


def _readout(reps, batch2, bs, ros):
    c = reps[0].shape[1]
    l1 = [p["lins"][0] for p in ros]
    l2 = [p["lins"][1] for p in ros]
    l3 = [p["lins"][2] for p in ros]
    n1 = [p["norms"][0] for p in ros]
    n2 = [p["norms"][1] for p in ros]
    t1s = _pool_mm(reps, batch2, bs, [w for w, _ in l1], [b for _, b in l1])
    st1 = [_stats(t) for t in t1s]
    t2s = _bn_mm4(t1s, st1, [g for g, _ in n1], [b for _, b in n1],
                  [w for w, _ in l2], [b for _, b in l2])
    st2 = [_stats(t) for t in t2s]
    return _bn_sum4(t2s, st2, [g for g, _ in n2], [b for _, b in n2],
                    [w for w, _ in l3], [b for _, b in l3])


def kernel(x, edge_index, edge_attr, batch, batch_size, params):
    n, c = x.shape
    e = edge_index.shape[1]

    per_w = -(-e // (NW * CHUNK))
    per_w += per_w % 2
    e_pad = NW * per_w * CHUNK
    rpt = -(-(-(-n // NS)) // 8) * 8
    aggr_rows = NS * rpt

    src_p = jnp.pad(edge_index[0], (0, e_pad - e))
    dst_p = jnp.pad(edge_index[1], (0, e_pad - e), constant_values=n)
    ea0 = jnp.pad(edge_attr[:, 0:1], ((0, e_pad - e), (0, 0)))
    ea1 = jnp.pad(edge_attr[:, 1:2], ((0, e_pad - e), (0, 0)))
    zeros_blk = jnp.zeros((rpt, c), jnp.float32)
    bs = 64
    batch2 = jnp.minimum(batch, batch_size - 1).reshape(1, n)

    h = x
    reps = [x]
    for cp in params["convs"]:
        we, be = cp["edge_lin"]
        wcat = jnp.zeros((8, c), jnp.float32)
        wcat = wcat.at[0].set(we[0]).at[1].set(we[1]).at[2].set(be)
        g = _gather_rows(h, src_p, per_w)
        msg = _edge_msg(g, ea0, ea1, wcat)
        aggr = _segment_add(msg, dst_p, zeros_blk, aggr_rows, per_w)
        h = _node_mlp(h, aggr, aggr_rows, cp)
        reps.append(h)

    return _readout(reps, batch2, bs, params["readouts"])


# speedup vs baseline: 1.4190x; 1.0382x over previous
import functools

import jax
import jax.numpy as jnp
from jax import lax
from jax.experimental import pallas as pl
from jax.experimental.pallas import tpu as pltpu
from jax.experimental.pallas import tpu_sc as plsc

NC = 2
NS = 16
NW = NC * NS
CHUNK = 128

_PREC = lax.Precision.HIGHEST
_MLP_PREC = lax.Precision.DEFAULT


def _gather_rows(h, src_p, per_w):
    e_pad = src_p.shape[0]
    n, c = h.shape
    mesh = plsc.VectorSubcoreMesh(core_axis_name="c", subcore_axis_name="s")

    @functools.partial(
        pl.kernel,
        out_type=jax.ShapeDtypeStruct((e_pad, c), jnp.float32),
        mesh=mesh,
        scratch_types=[
            pltpu.VMEM((CHUNK, c), jnp.float32),
            pltpu.VMEM((CHUNK, c), jnp.float32),
            pltpu.VMEM((CHUNK,), jnp.int32),
            pltpu.VMEM((CHUNK,), jnp.int32),
            pltpu.SemaphoreType.DMA,
            pltpu.SemaphoreType.DMA,
            pltpu.SemaphoreType.DMA,
            pltpu.SemaphoreType.DMA,
            pltpu.SemaphoreType.DMA,
        ],
    )
    def k(h_hbm, s_hbm, g_hbm, rb0, rb1, ib0, ib1, si0, si1, sg, so0, so1):
        w = lax.axis_index("s") * NC + lax.axis_index("c")
        base = w * per_w * CHUNK
        rb = (rb0, rb1)
        ib = (ib0, ib1)
        si = (si0, si1)
        so = (so0, so1)

        def idx_copy(b, kk):
            return pltpu.make_async_copy(
                s_hbm.at[pl.ds(base + kk * CHUNK, CHUNK)], ib[b], si[b])

        def out_copy(b, kk):
            return pltpu.make_async_copy(
                rb[b], g_hbm.at[pl.ds(base + kk * CHUNK, CHUNK)], so[b])

        idx_copy(0, 0).start()
        idx_copy(1, 1).start()

        @pl.loop(0, per_w, step=2)
        def _(k0):
            for b in range(2):
                kk = k0 + b
                idx_copy(b, kk).wait()

                @pl.when(kk >= 2)
                def _():
                    out_copy(b, kk).wait()

                pltpu.async_copy(h_hbm.at[ib[b]], rb[b], sg).wait()
                out_copy(b, kk).start()

                @pl.when(kk + 2 < per_w)
                def _():
                    idx_copy(b, kk + 2).start()

        out_copy(0, 0).wait()
        out_copy(1, 1).wait()

    return k(h, src_p)


def _segment_add(msg, dst_p, zeros_blk, aggr_rows, per_w):
    e_pad = dst_p.shape[0]
    c_dim = msg.shape[1]
    rpt = aggr_rows // NS
    mesh = plsc.VectorSubcoreMesh(core_axis_name="c", subcore_axis_name="s")

    @functools.partial(
        pl.kernel,
        out_type=jax.ShapeDtypeStruct((2 * aggr_rows, c_dim), jnp.float32),
        mesh=mesh,
        scratch_types=[
            pltpu.VMEM((CHUNK, c_dim), jnp.float32),
            pltpu.VMEM((CHUNK, c_dim), jnp.float32),
            pltpu.VMEM((CHUNK,), jnp.int32),
            pltpu.VMEM((CHUNK,), jnp.int32),
            pltpu.VMEM_SHARED((aggr_rows, c_dim), jnp.float32),
            pltpu.SemaphoreType.DMA,
            pltpu.SemaphoreType.DMA,
            pltpu.SemaphoreType.DMA,
            pltpu.SemaphoreType.DMA,
            pltpu.SemaphoreType.DMA,
        ],
    )
    def k(m_hbm, d_hbm, z_hbm, o_hbm, mb0, mb1, ib0, ib1, acc,
          sm0, sm1, si0, si1, sz):
        cc = lax.axis_index("c")
        ss = lax.axis_index("s")
        w = ss * NC + cc
        base = w * per_w * CHUNK
        r0 = ss * rpt
        pltpu.async_copy(z_hbm, acc.at[pl.ds(r0, rpt)], sz).wait()
        plsc.subcore_barrier()
        mb = (mb0, mb1)
        ib = (ib0, ib1)
        sm = (sm0, sm1)
        si = (si0, si1)

        def msg_copy(b, kk):
            return pltpu.make_async_copy(
                m_hbm.at[pl.ds(base + kk * CHUNK, CHUNK)], mb[b], sm[b])

        def idx_copy(b, kk):
            return pltpu.make_async_copy(
                d_hbm.at[pl.ds(base + kk * CHUNK, CHUNK)], ib[b], si[b])

        msg_copy(0, 0).start()
        idx_copy(0, 0).start()
        msg_copy(1, 1).start()
        idx_copy(1, 1).start()

        @pl.loop(0, per_w, step=2)
        def _(k0):
            for b in range(2):
                kk = k0 + b
                msg_copy(b, kk).wait()
                idx_copy(b, kk).wait()
                pltpu.sync_copy(mb[b], acc.at[ib[b]], add=True)

                @pl.when(kk + 2 < per_w)
                def _():
                    msg_copy(b, kk + 2).start()
                    idx_copy(b, kk + 2).start()

        plsc.subcore_barrier()
        pltpu.sync_copy(acc.at[pl.ds(r0, rpt)],
                        o_hbm.at[pl.ds(cc * aggr_rows + r0, rpt)])

    return k(msg, dst_p, zeros_blk)


def _edge_msg(g, ea0, ea1, wcat):
    e_pad, c = g.shape
    blk = 1024
    grid = e_pad // blk

    def body(g_ref, a0_ref, a1_ref, w_ref, o_ref):
        w = w_ref[...]
        t = (a0_ref[...] * w[0:1, :] + a1_ref[...] * w[1:2, :]) + g_ref[...]
        o_ref[...] = jnp.maximum(t + w[2:3, :], 0.0)

    return pl.pallas_call(
        body,
        grid=(grid,),
        in_specs=[
            pl.BlockSpec((blk, c), lambda i: (i, 0)),
            pl.BlockSpec((blk, 1), lambda i: (i, 0)),
            pl.BlockSpec((blk, 1), lambda i: (i, 0)),
            pl.BlockSpec((8, c), lambda i: (0, 0)),
        ],
        out_specs=pl.BlockSpec((blk, c), lambda i: (i, 0)),
        out_shape=jax.ShapeDtypeStruct((e_pad, c), jnp.float32),
    )(g, ea0, ea1, wcat)


def _bn_relu(t, gamma, beta):
    mu = jnp.mean(t, axis=0, keepdims=True)
    var = jnp.var(t, axis=0, keepdims=True)
    return jnp.maximum((t - mu) / jnp.sqrt(var + 1e-5) * gamma + beta, 0.0)


def _node_mlp(h, aggr, aggr_rows, cp):
    n, c = h.shape
    onep = (1.0 + cp["eps"]).reshape(1, 1).astype(jnp.float32)
    (w1, b1), (w2, b2), (w3, b3) = cp["mlp"]["lins"]
    (g1, be1), (g2, be2) = cp["mlp"]["norms"]

    def body(h_ref, ag_ref, onep_ref, w1_ref, b1_ref, g1_ref, be1_ref,
             w2_ref, b2_ref, g2_ref, be2_ref, w3_ref, b3_ref, o_ref):
        a = ag_ref[0:n, :] + ag_ref[aggr_rows:aggr_rows + n, :]
        u = onep_ref[...] * h_ref[...] + a
        t = jnp.dot(u, w1_ref[...], precision=_MLP_PREC,
                    preferred_element_type=jnp.float32) + b1_ref[...]
        t = _bn_relu(t, g1_ref[...], be1_ref[...])
        t = jnp.dot(t, w2_ref[...], precision=_MLP_PREC,
                    preferred_element_type=jnp.float32) + b2_ref[...]
        t = _bn_relu(t, g2_ref[...], be2_ref[...])
        t = jnp.dot(t, w3_ref[...], precision=_MLP_PREC,
                    preferred_element_type=jnp.float32) + b3_ref[...]
        o_ref[...] = jnp.maximum(t, 0.0)

    return pl.pallas_call(
        body,
        out_shape=jax.ShapeDtypeStruct((n, c), jnp.float32),
    )(h, aggr, onep,
      w1, b1.reshape(1, c), g1.reshape(1, c), be1.reshape(1, c),
      w2, b2.reshape(1, c), g2.reshape(1, c), be2.reshape(1, c),
      w3, b3.reshape(1, c))


def _readout(reps, batch2, bs, ros):
    n, c = reps[0].shape
    flat = []
    for p in ros:
        (w1, b1), (w2, b2), (w3, b3) = p["lins"]
        (g1, be1), (g2, be2) = p["norms"]
        flat += [w1, b1.reshape(1, c), g1.reshape(1, c), be1.reshape(1, c),
                 w2, b2.reshape(1, c), g2.reshape(1, c), be2.reshape(1, c),
                 w3, b3.reshape(1, c)]

    def body(*refs):
        r_refs = refs[:4]
        b_ref = refs[4]
        p_refs = refs[5:5 + 40]
        o_ref = refs[-1]
        ids = b_ref[...]
        oht = (lax.broadcasted_iota(jnp.int32, (bs, n), 0) == ids
               ).astype(jnp.float32)
        acc = None
        for l in range(4):
            pooled = jnp.dot(oht, r_refs[l][...], precision=_PREC,
                             preferred_element_type=jnp.float32)
            pw = p_refs[l * 10:(l + 1) * 10]
            t = jnp.dot(pooled, pw[0][...], precision=_MLP_PREC,
                        preferred_element_type=jnp.float32) + pw[1][...]
            t = _bn_relu(t, pw[2][...], pw[3][...])
            t = jnp.dot(t, pw[4][...], precision=_MLP_PREC,
                        preferred_element_type=jnp.float32) + pw[5][...]
            t = _bn_relu(t, pw[6][...], pw[7][...])
            t = jnp.dot(t, pw[8][...], precision=_MLP_PREC,
                        preferred_element_type=jnp.float32) + pw[9][...]
            acc = t if acc is None else acc + t
        o_ref[...] = acc

    return pl.pallas_call(
        body,
        out_shape=jax.ShapeDtypeStruct((bs, c), jnp.float32),
    )(*reps, batch2, *flat)


def kernel(x, edge_index, edge_attr, batch, batch_size, params):
    n, c = x.shape
    e = edge_index.shape[1]

    per_w = -(-e // (NW * CHUNK))
    per_w += per_w % 2
    e_pad = NW * per_w * CHUNK
    rpt = -(-(-(-n // NS)) // 8) * 8
    aggr_rows = NS * rpt

    src_p = jnp.pad(edge_index[0], (0, e_pad - e))
    dst_p = jnp.pad(edge_index[1], (0, e_pad - e), constant_values=n)
    ea0 = jnp.pad(edge_attr[:, 0:1], ((0, e_pad - e), (0, 0)))
    ea1 = jnp.pad(edge_attr[:, 1:2], ((0, e_pad - e), (0, 0)))
    zeros_blk = jnp.zeros((rpt, c), jnp.float32)
    bs = 64
    batch2 = jnp.minimum(batch, batch_size - 1).reshape(1, n)

    h = x
    reps = [x]
    for cp in params["convs"]:
        we, be = cp["edge_lin"]
        wcat = jnp.zeros((8, c), jnp.float32)
        wcat = wcat.at[0].set(we[0]).at[1].set(we[1]).at[2].set(be)
        g = _gather_rows(h, src_p, per_w)
        msg = _edge_msg(g, ea0, ea1, wcat)
        aggr = _segment_add(msg, dst_p, zeros_blk, aggr_rows, per_w)
        h = _node_mlp(h, aggr, aggr_rows, cp)
        reps.append(h)

    return _readout(reps, batch2, bs, params["readouts"])
